# baseline (speedup 1.0000x reference)
import jax
import jax.numpy as jnp
from jax.experimental import pallas as pl
from jax.experimental.pallas import tpu as pltpu

_ROWS = 16384
_D = 2048
_CHUNK = 1024
_NCHUNK = _ROWS // _CHUNK
_NBUF = 4


def _copy_body(x_ref, o_ref, *rest):
    bufs = rest[:_NBUF]
    rsems = rest[_NBUF:2 * _NBUF]
    wsems = rest[2 * _NBUF:3 * _NBUF]
    reads = [None] * _NBUF
    writes = [None] * _NBUF
    for g in range(_NBUF - 1):
        reads[g] = pltpu.make_async_copy(
            x_ref.at[pl.ds(g * _CHUNK, _CHUNK)], bufs[g], rsems[g])
        reads[g].start()
    for g in range(_NCHUNK):
        b = g % _NBUF
        reads[b].wait()
        writes[b] = pltpu.make_async_copy(
            bufs[b], o_ref.at[pl.ds(g * _CHUNK, _CHUNK)], wsems[b])
        writes[b].start()
        nxt = g + _NBUF - 1
        if nxt < _NCHUNK:
            nb = nxt % _NBUF
            if writes[nb] is not None:
                writes[nb].wait()
            reads[nb] = pltpu.make_async_copy(
                x_ref.at[pl.ds(nxt * _CHUNK, _CHUNK)], bufs[nb], rsems[nb])
            reads[nb].start()
    for b in range(_NBUF):
        if writes[b] is not None:
            writes[b].wait()


def kernel(x, rel_pos_bias):
    del rel_pos_bias
    b, s, d = x.shape
    x2 = x.reshape(b * s, d)
    out = pl.pallas_call(
        _copy_body,
        out_shape=jax.ShapeDtypeStruct((b * s, d), x.dtype),
        in_specs=[pl.BlockSpec(memory_space=pl.ANY)],
        out_specs=pl.BlockSpec(memory_space=pl.ANY),
        scratch_shapes=(
            [pltpu.VMEM((_CHUNK, _D), jnp.float32)] * _NBUF
            + [pltpu.SemaphoreType.DMA] * (2 * _NBUF)
        ),
    )(x2)
    return out.reshape(b, s, d)


# speedup vs baseline: 1.0022x; 1.0022x over previous
import functools

import jax
import jax.numpy as jnp
from jax.experimental import pallas as pl
from jax.experimental.pallas import tpu as pltpu

_MAX_CHUNK_ROWS = 1024


def _copy_body(x_ref, o_ref, *rest, chunk, nchunk, nbuf):
    bufs = rest[:nbuf]
    rsems = rest[nbuf:2 * nbuf]
    wsems = rest[2 * nbuf:3 * nbuf]
    reads = [None] * nbuf
    writes = [None] * nbuf
    for g in range(min(nbuf - 1, nchunk)):
        reads[g] = pltpu.make_async_copy(
            x_ref.at[pl.ds(g * chunk, chunk)], bufs[g], rsems[g])
        reads[g].start()
    for g in range(nchunk):
        b = g % nbuf
        reads[b].wait()
        writes[b] = pltpu.make_async_copy(
            bufs[b], o_ref.at[pl.ds(g * chunk, chunk)], wsems[b])
        writes[b].start()
        nxt = g + nbuf - 1
        if nxt < nchunk:
            nb = nxt % nbuf
            if writes[nb] is not None:
                writes[nb].wait()
            reads[nb] = pltpu.make_async_copy(
                x_ref.at[pl.ds(nxt * chunk, chunk)], bufs[nb], rsems[nb])
            reads[nb].start()
    for b in range(nbuf):
        if writes[b] is not None:
            writes[b].wait()


def kernel(x, rel_pos_bias):
    del rel_pos_bias
    b, s, d = x.shape
    rows = b * s
    x2 = x.reshape(rows, d)
    chunk = 1
    while chunk < _MAX_CHUNK_ROWS and rows % (chunk * 2) == 0:
        chunk *= 2
    nchunk = rows // chunk
    nbuf = min(3, nchunk)
    body = functools.partial(_copy_body, chunk=chunk, nchunk=nchunk, nbuf=nbuf)
    out = pl.pallas_call(
        body,
        out_shape=jax.ShapeDtypeStruct((rows, d), x.dtype),
        in_specs=[pl.BlockSpec(memory_space=pl.ANY)],
        out_specs=pl.BlockSpec(memory_space=pl.ANY),
        scratch_shapes=(
            [pltpu.VMEM((chunk, d), x.dtype)] * nbuf
            + [pltpu.SemaphoreType.DMA] * (2 * nbuf)
        ),
    )(x2)
    return out.reshape(b, s, d)
